# baseline (speedup 1.0000x reference)
import functools

import jax
import jax.numpy as jnp
from jax import lax
from jax.experimental import pallas as pl
from jax.experimental.pallas import tpu as pltpu
from jax.experimental.pallas import tpu_sc as plsc

S, H, I = 2048, 768, 384
E = 7
EP = 128
NEG = -1e30
TILE = 128
NA = 2 * S
NT_SH = S // TILE
NT_RT = NA // TILE + E
NT = NT_SH + NT_RT
N_XS = NT_RT * TILE

NC, NS = 2, 16
NW = NC * NS
APW = NA // NW
TPW = S // NW


def _router_body(xr, wrr, rbr, sc_out, idx_out):
    probs = jax.nn.sigmoid(xr[...] @ wrr[...] + rbr[...])
    lane = lax.broadcasted_iota(jnp.int32, (S, EP), 1)
    m0 = jnp.max(probs, axis=1, keepdims=True)
    i0 = jnp.min(jnp.where(probs == m0, lane, EP), axis=1, keepdims=True)
    probs1 = jnp.where(lane == i0, NEG, probs)
    m1 = jnp.max(probs1, axis=1, keepdims=True)
    i1 = jnp.min(jnp.where(probs1 == m1, lane, EP), axis=1, keepdims=True)
    lane32 = lax.broadcasted_iota(jnp.int32, (S, 32), 1)
    sc_out[...] = jnp.where(lane32 < 16, m0, m1)
    lane8 = lax.broadcasted_iota(jnp.int32, (S, 8), 1)
    idx_out[...] = jnp.where(lane8 == 0, i0, jnp.where(lane8 == 1, i1, 0))


def _router(xf, Wr, rbias):
    Wrp = jnp.zeros((H, EP), jnp.float32).at[:, :E].set(Wr)
    rbp = jnp.full((1, EP), NEG, jnp.float32).at[0, :E].set(rbias)
    return pl.pallas_call(
        _router_body,
        in_specs=[
            pl.BlockSpec((S, H), lambda: (0, 0)),
            pl.BlockSpec((H, EP), lambda: (0, 0)),
            pl.BlockSpec((1, EP), lambda: (0, 0)),
        ],
        out_specs=[
            pl.BlockSpec((S, 32), lambda: (0, 0)),
            pl.BlockSpec((S, 8), lambda: (0, 0)),
        ],
        out_shape=[
            jax.ShapeDtypeStruct((S, 32), jnp.float32),
            jax.ShapeDtypeStruct((S, 8), jnp.int32),
        ],
    )(xf, Wrp, rbp)


def _dispatch_body(x_hbm, tok_hbm, pos_hbm, xs_hbm, tok_v, pos_v, rows_v, sem1, sem2):
    wid = lax.axis_index("s") * NC + lax.axis_index("c")
    base = wid * APW
    pltpu.sync_copy(tok_hbm.at[pl.ds(base, APW)], tok_v)
    pltpu.sync_copy(pos_hbm.at[pl.ds(base, APW)], pos_v)
    pltpu.async_copy(x_hbm.at[tok_v], rows_v, sem1).wait()
    pltpu.async_copy(rows_v, xs_hbm.at[pos_v], sem2).wait()


def _dispatch(xf, tok, pos_xs):
    mesh = plsc.VectorSubcoreMesh(core_axis_name="c", subcore_axis_name="s")
    k = pl.kernel(
        _dispatch_body,
        mesh=mesh,
        out_type=jax.ShapeDtypeStruct((N_XS, H), jnp.float32),
        scratch_types=[
            pltpu.VMEM((APW,), jnp.int32),
            pltpu.VMEM((APW,), jnp.int32),
            pltpu.VMEM((APW, H), jnp.float32),
            pltpu.SemaphoreType.DMA,
            pltpu.SemaphoreType.DMA,
        ],
    )
    return k(xf, tok, pos_xs)


def _gemm_body(te_ref, act_ref, xr, xsr, wgr, wur, wdr, ysr):
    i = pl.program_id(0)

    @pl.when(act_ref[i] == 1)
    def _():
        bf = jnp.bfloat16
        src = jnp.where(i < NT_SH, xr[...], xsr[...]).astype(bf)
        wg = wgr[0].astype(bf)
        wu = wur[0].astype(bf)
        wd = wdr[0].astype(bf)
        mm = functools.partial(lax.dot, preferred_element_type=jnp.float32)
        h = jax.nn.silu(mm(src, wg)) * mm(src, wu)
        ysr[...] = mm(h.astype(bf), wd)


def _grouped_gemm(xf, xs, Wg_all, Wu_all, Wd_all, tile_expert, active):
    grid_spec = pltpu.PrefetchScalarGridSpec(
        num_scalar_prefetch=2,
        grid=(NT,),
        in_specs=[
            pl.BlockSpec((TILE, H), lambda i, te, act: (jnp.minimum(i, NT_SH - 1), 0)),
            pl.BlockSpec((TILE, H),
                         lambda i, te, act: (jnp.where(act[i] == 1,
                                                       jnp.maximum(i - NT_SH, 0), 0), 0)),
            pl.BlockSpec((1, H, I), lambda i, te, act: (te[i], 0, 0)),
            pl.BlockSpec((1, H, I), lambda i, te, act: (te[i], 0, 0)),
            pl.BlockSpec((1, I, H), lambda i, te, act: (te[i], 0, 0)),
        ],
        out_specs=pl.BlockSpec((TILE, H), lambda i, te, act: (i, 0)),
    )
    return pl.pallas_call(
        _gemm_body,
        grid_spec=grid_spec,
        out_shape=jax.ShapeDtypeStruct((NT * TILE, H), jnp.float32),
        compiler_params=pltpu.CompilerParams(
            dimension_semantics=("arbitrary",),
        ),
    )(tile_expert, active, xf, xs, Wg_all, Wu_all, Wd_all)


_CH = 32


def _combine_body(ys_hbm, p0_hbm, p1_hbm, sc_hbm, out_hbm,
                  acc_v, r0_v, r1_v, s_v, p0_v, p1_v, sem0, sem1):
    wid = lax.axis_index("s") * NC + lax.axis_index("c")
    for half in range(TPW // _CH):
        tb = wid * TPW + half * _CH
        pltpu.sync_copy(p0_hbm.at[pl.ds(tb, _CH)], p0_v)
        pltpu.sync_copy(p1_hbm.at[pl.ds(tb, _CH)], p1_v)
        g0 = pltpu.async_copy(ys_hbm.at[p0_v], r0_v, sem0)
        g1 = pltpu.async_copy(ys_hbm.at[p1_v], r1_v, sem1)
        pltpu.sync_copy(ys_hbm.at[pl.ds(tb, _CH)], acc_v)
        pltpu.sync_copy(sc_hbm.at[pl.ds(tb, _CH)], s_v)
        g0.wait()
        g1.wait()

        def body(j, _):
            s0 = s_v[j, pl.ds(0, 16)]
            s1 = s_v[j, pl.ds(16, 16)]
            for c in range(H // 16):
                sl = pl.ds(c * 16, 16)
                acc_v[j, sl] = acc_v[j, sl] + s0 * r0_v[j, sl] + s1 * r1_v[j, sl]
            return 0

        lax.fori_loop(0, _CH, body, 0)
        pltpu.sync_copy(acc_v, out_hbm.at[pl.ds(tb, _CH)])


def _combine(ys, p0, p1, scores):
    mesh = plsc.VectorSubcoreMesh(core_axis_name="c", subcore_axis_name="s")
    k = pl.kernel(
        _combine_body,
        mesh=mesh,
        out_type=jax.ShapeDtypeStruct((S, H), jnp.float32),
        scratch_types=[
            pltpu.VMEM((_CH, H), jnp.float32),
            pltpu.VMEM((_CH, H), jnp.float32),
            pltpu.VMEM((_CH, H), jnp.float32),
            pltpu.VMEM((_CH, 32), jnp.float32),
            pltpu.VMEM((_CH,), jnp.int32),
            pltpu.VMEM((_CH,), jnp.int32),
            pltpu.SemaphoreType.DMA,
            pltpu.SemaphoreType.DMA,
        ],
    )
    return k(ys, p0, p1, scores)


def kernel(x, Wg_s, Wu_s, Wd_s, Wg, Wu, Wd, Wr, rbias):
    xf = x.reshape(S, H)
    scores, idx2 = _router(xf, Wr, rbias)

    i0 = jnp.clip(idx2[:, 0], 0, E - 1)
    i1 = jnp.clip(idx2[:, 1], 0, E - 1)
    eflat = jnp.concatenate([i0, i1])
    oh = jax.nn.one_hot(eflat, E, dtype=jnp.int32)
    cs = jnp.cumsum(oh, axis=0)
    rank = jnp.take_along_axis(cs - oh, eflat[:, None], axis=1)[:, 0]
    counts = cs[-1]
    tiles_e = (counts + TILE - 1) // TILE
    cumt = jnp.cumsum(tiles_e)
    tile_base = jnp.concatenate([jnp.zeros((1,), jnp.int32), cumt[:-1]])
    pos_xs = (tile_base[eflat] * TILE + rank).astype(jnp.int32)
    pos_ys = pos_xs + S
    n_rt = cumt[E - 1]
    j = jnp.arange(NT_RT, dtype=jnp.int32)
    act_r = (j < n_rt).astype(jnp.int32)
    texp_r = jnp.clip(jnp.searchsorted(cumt, j, side="right"), 0, E - 1)
    texp_r = jnp.where(act_r == 1, texp_r, 0).astype(jnp.int32)
    tile_expert = jnp.concatenate(
        [jnp.full((NT_SH,), E, jnp.int32), texp_r])
    active = jnp.concatenate([jnp.ones((NT_SH,), jnp.int32), act_r])
    tok = jnp.tile(jnp.arange(S, dtype=jnp.int32), 2)

    xs = _dispatch(xf, tok, pos_xs)

    Wg_all = jnp.concatenate([Wg, Wg_s[None]], axis=0)
    Wu_all = jnp.concatenate([Wu, Wu_s[None]], axis=0)
    Wd_all = jnp.concatenate([Wd, Wd_s[None]], axis=0)
    ys = _grouped_gemm(xf, xs, Wg_all, Wu_all, Wd_all, tile_expert, active)

    p0 = pos_ys[:S]
    p1 = pos_ys[S:]
    out = _combine(ys, p0, p1, scores)
    return out.reshape(1, S, H)
